# baseline (speedup 1.0000x reference)
import functools

import jax
import jax.numpy as jnp
from jax import lax
from jax.experimental import pallas as pl
from jax.experimental.pallas import tpu as pltpu
from jax.experimental.pallas import tpu_sc as plsc

N_NODES = 100000
D = 128
M = 200000
K = 3

_LANES = 16
_B = 128


def _build(nc: int, ns: int):
    nw = nc * ns
    w_lo = ((M // nw) // 8) * 8
    extra = M - nw * w_lo
    assert extra % 8 == 0 and extra // 8 <= nw
    n_hi = extra // 8
    t0 = nw - n_hi
    w_hi = w_lo + 8
    n_full = w_lo // _B
    hi_tail = w_hi - n_full * _B
    tb = -(-hi_tail // _LANES) * _LANES
    assert 0 < tb <= _B and tb <= n_full * _B and tb % 8 == 0
    assert n_full >= 8

    mesh = plsc.VectorSubcoreMesh(core_axis_name="c", subcore_axis_name="s")

    @functools.partial(
        pl.kernel,
        mesh=mesh,
        compiler_params=pltpu.CompilerParams(use_tc_tiling_on_sc=False),
        out_type=jax.ShapeDtypeStruct((M, D), jnp.float32),
        scratch_types=(
            [pltpu.VMEM((_B, D), jnp.float32) for _ in range(6)]
            + [pltpu.VMEM((K, _B), jnp.int32) for _ in range(4)]
            + [pltpu.SemaphoreType.DMA for _ in range(8)]
        ),
    )
    def sc_kernel(x_hbm, idx_hbm, out_hbm, *refs):
        wid = lax.axis_index("s") * nc + lax.axis_index("c")
        base = wid * w_lo + 8 * jnp.maximum(wid - t0, 0)
        mpw = w_lo + 8 * (wid >= t0).astype(jnp.int32)

        bufs, cols, sems = refs[:6], refs[6:10], refs[10:]
        dsets = [(bufs[3 * p], bufs[3 * p + 1], bufs[3 * p + 2],
                  sems[2 * p], sems[2 * p + 1]) for p in range(2)]
        isets = [(cols[p], sems[4 + p]) for p in range(4)]
        S = lambda j: dsets[j % 2]
        I = lambda j: isets[j % 4]

        def row0(c):
            if isinstance(c, int) and c == n_full:
                return mpw - tb
            return c * _B

        def idx_copies(c, iset, n):
            col, semi = iset
            r0 = base + row0(c)
            return [
                pltpu.make_async_copy(
                    idx_hbm.at[k, pl.ds(r0, n)],
                    col.at[k] if n == _B else col.at[k, pl.ds(0, n)],
                    semi)
                for k in range(K)
            ]

        def gather_copies(st, iset, n):
            col = iset[0]
            ob = st[0]
            dst = ob if n == _B else ob.at[pl.ds(0, n)]
            return [
                pltpu.make_async_copy(
                    x_hbm.at[col.at[k] if n == _B else col.at[k, pl.ds(0, n)]],
                    dst, st[3])
                for k in range(K)
            ]

        def fire_adds(st, iset, n):
            col = iset[0]
            ob = st[0]
            dst = ob if n == _B else ob.at[pl.ds(0, n)]
            for k in range(1, K):
                src = col.at[k] if n == _B else col.at[k, pl.ds(0, n)]
                pltpu.async_copy(x_hbm.at[src], dst, st[3], add=True)

        def store_copy(c, st, n):
            src = st[0] if n == _B else st[0].at[pl.ds(0, n)]
            return pltpu.make_async_copy(
                src, out_hbm.at[pl.ds(base + row0(c), n)], st[4])

        def fire(descs):
            for d in descs:
                d.start()

        def drain(descs):
            for d in descs:
                d.wait()

        def accumulate(st, n=_B):
            ob, b1, b2 = st[0], st[1], st[2]

            def rows(r2, carry):
                for u in range(2):
                    r = r2 * 2 + u
                    for j in range(D // _LANES):
                        sl = pl.ds(j * _LANES, _LANES)
                        plsc.addupdate(ob.at[r, sl], b1[r, sl] + b2[r, sl])
                return carry

            lax.fori_loop(0, n // 2, rows, 0)

        def steady(c, cm, n2=_B):
            cur, oth = S(cm), S(cm + 1)
            drain(idx_copies(c + 1, I(cm + 1), _B))
            drain([store_copy(c - 1, oth, _B)])
            fire(gather_copies(oth, I(cm + 1), _B)[:1])
            fire(idx_copies(c + 2 if n2 == _B else n_full, I(cm + 2), n2))
            drain(gather_copies(cur, I(cm), _B)[:1])
            fire_adds(cur, I(cm), _B)
            drain(gather_copies(cur, I(cm), _B)[1:])
            fire([store_copy(c, cur, _B)])

        fire(idx_copies(0, I(0), _B))
        fire(idx_copies(1, I(1), _B))
        drain(idx_copies(0, I(0), _B))
        fire(gather_copies(S(0), I(0), _B)[:1])
        fire(idx_copies(2, I(2), _B))
        drain(idx_copies(1, I(1), _B))
        fire(gather_copies(S(1), I(1), _B)[:1])
        fire(idx_copies(3, I(3), _B))
        drain(gather_copies(S(0), I(0), _B)[:1])
        fire_adds(S(0), I(0), _B)
        drain(gather_copies(S(0), I(0), _B)[1:])
        fire([store_copy(0, S(0), _B)])

        n_steady = n_full - 2
        def body(m, carry):
            for i in range(4):
                steady(4 * m + 1 + i, 1 + i)
            return carry
        lax.fori_loop(0, (n_steady - 1) // 4, body, 0)
        for j in range(4 * ((n_steady - 1) // 4) + 1, n_steady + 1):
            steady(j, j, n2=(_B if j + 2 <= n_full - 1 else tb))

        c = n_full - 1
        drain(idx_copies(n_full, I(n_full), tb))
        drain([store_copy(c - 1, S(c + 1), _B)])
        fire(gather_copies(S(c + 1), I(n_full), tb)[:1])
        drain(gather_copies(S(c), I(c), _B)[:1])
        fire_adds(S(c), I(c), _B)
        drain(gather_copies(S(c), I(c), _B)[1:])
        fire([store_copy(c, S(c), _B)])

        st = S(n_full)
        drain(gather_copies(st, I(n_full), tb)[:1])
        fire_adds(st, I(n_full), tb)
        drain(gather_copies(st, I(n_full), tb)[1:])
        drain([store_copy(n_full - 1, S(n_full - 1), _B)])
        pltpu.sync_copy(st[0].at[pl.ds(0, tb)],
                        out_hbm.at[pl.ds(base + row0(n_full), tb)])

    def run(x, indices):
        return sc_kernel(x, indices.astype(jnp.int32).T)

    return run


def kernel(x, indices):
    info = plsc.get_sparse_core_info()
    return _build(info.num_cores, info.num_subcores)(x, indices)


# speedup vs baseline: 1.1569x; 1.1569x over previous
import functools

import jax
import jax.numpy as jnp
from jax import lax
from jax.experimental import pallas as pl
from jax.experimental.pallas import tpu as pltpu
from jax.experimental.pallas import tpu_sc as plsc

N_NODES = 100000
D = 128
M = 200000
K = 3

_LANES = 16
_B = 128
_NS = 4


def _build(nc: int, ns: int):
    nw = nc * ns
    w_lo = ((M // nw) // 8) * 8
    extra = M - nw * w_lo
    assert extra % 8 == 0 and extra // 8 <= nw
    n_hi = extra // 8
    t0 = nw - n_hi
    w_hi = w_lo + 8
    n_full = w_lo // _B
    hi_tail = w_hi - n_full * _B
    tb = -(-hi_tail // _LANES) * _LANES
    assert 0 < tb <= _B and tb <= n_full * _B and tb % 8 == 0
    assert n_full >= 3 * _NS

    mesh = plsc.VectorSubcoreMesh(core_axis_name="c", subcore_axis_name="s")

    @functools.partial(
        pl.kernel,
        mesh=mesh,
        compiler_params=pltpu.CompilerParams(use_tc_tiling_on_sc=False),
        out_type=jax.ShapeDtypeStruct((M, D), jnp.float32),
        scratch_types=(
            [pltpu.VMEM((_B, D), jnp.float32) for _ in range(_NS)]
            + [pltpu.VMEM((K, _B), jnp.int32) for _ in range(_NS)]
            + [pltpu.SemaphoreType.DMA for _ in range(3 * _NS)]
        ),
    )
    def sc_kernel(x_hbm, idx_hbm, out_hbm, *refs):
        wid = lax.axis_index("s") * nc + lax.axis_index("c")
        base = wid * w_lo + 8 * jnp.maximum(wid - t0, 0)
        mpw = w_lo + 8 * (wid >= t0).astype(jnp.int32)

        obs, cols = refs[:_NS], refs[_NS:2 * _NS]
        sems = refs[2 * _NS:]
        dsets = [(obs[p], sems[2 * p], sems[2 * p + 1]) for p in range(_NS)]
        isets = [(cols[p], sems[2 * _NS + p]) for p in range(_NS)]
        S = lambda j: dsets[j % _NS]
        I = lambda j: isets[j % _NS]

        def row0(c):
            if isinstance(c, int) and c == n_full:
                return mpw - tb
            return c * _B

        def idx_copies(c, iset, n):
            col, semi = iset
            r0 = base + row0(c)
            return [
                pltpu.make_async_copy(
                    idx_hbm.at[k, pl.ds(r0, n)],
                    col.at[k] if n == _B else col.at[k, pl.ds(0, n)],
                    semi)
                for k in range(K)
            ]

        def gather_copies(st, iset, n):
            col = iset[0]
            dst = st[0] if n == _B else st[0].at[pl.ds(0, n)]
            return [
                pltpu.make_async_copy(
                    x_hbm.at[col.at[k] if n == _B else col.at[k, pl.ds(0, n)]],
                    dst, st[1])
                for k in range(K)
            ]

        def fire_adds(st, iset, n):
            col = iset[0]
            dst = st[0] if n == _B else st[0].at[pl.ds(0, n)]
            for k in range(1, K):
                src = col.at[k] if n == _B else col.at[k, pl.ds(0, n)]
                pltpu.async_copy(x_hbm.at[src], dst, st[1], add=True)

        def store_copy(c, st, n):
            src = st[0] if n == _B else st[0].at[pl.ds(0, n)]
            return pltpu.make_async_copy(
                src, out_hbm.at[pl.ds(base + row0(c), n)], st[2])

        def fire(descs):
            for d in descs:
                d.start()

        def drain(descs):
            for d in descs:
                d.wait()

        def step(c, cm, adds_prev=True, store_prev=True, drain_store=True,
                 i2=True, i2n=_B, i3=True, i3n=_B, k0n=_B):
            if adds_prev:
                drain(gather_copies(S(cm - 1), I(cm - 1), _B)[1:])
            if store_prev:
                fire([store_copy(c - 1, S(cm - 1), _B)])
            if drain_store:
                drain([store_copy(c - 2, S(cm - 2), _B)])
            if i2:
                c2 = n_full if i2n != _B else c + 2
                drain(idx_copies(c2, I(cm + 2), i2n))
                fire(gather_copies(S(cm + 2), I(cm + 2), i2n)[:1])
            if i3:
                c3 = n_full if i3n != _B else c + 3
                fire(idx_copies(c3, I(cm + 3), i3n))
            drain(gather_copies(S(cm), I(cm), k0n)[:1])
            fire_adds(S(cm), I(cm), k0n)

        fire(idx_copies(0, I(0), _B))
        fire(idx_copies(1, I(1), _B))
        fire(idx_copies(2, I(2), _B))
        drain(idx_copies(0, I(0), _B))
        fire(gather_copies(S(0), I(0), _B)[:1])
        drain(idx_copies(1, I(1), _B))
        fire(gather_copies(S(1), I(1), _B)[:1])

        step(0, 0, adds_prev=False, store_prev=False, drain_store=False)
        step(1, 1, drain_store=False)
        step(2, 2)

        lo = 3
        hi = n_full - 4
        n_blocks = (hi - lo) // _NS
        def body(m, carry):
            for i in range(_NS):
                step(lo + _NS * m + i, (lo + i) % _NS)
            return carry
        lax.fori_loop(0, n_blocks, body, 0)
        for c in range(lo + _NS * n_blocks, hi):
            step(c, c % _NS)

        step(n_full - 4, (n_full - 4) % _NS)
        step(n_full - 3, (n_full - 3) % _NS, i3n=tb)
        step(n_full - 2, (n_full - 2) % _NS, i2n=tb, i3=False)
        step(n_full - 1, (n_full - 1) % _NS, i2=False, i3=False)
        step(n_full, n_full % _NS, i2=False, i3=False, k0n=tb)

        st = S(n_full)
        drain(gather_copies(st, I(n_full), tb)[1:])
        drain([store_copy(n_full - 1, S(n_full - 1), _B)])
        pltpu.sync_copy(st[0].at[pl.ds(0, tb)],
                        out_hbm.at[pl.ds(base + row0(n_full), tb)])

    def run(x, indices):
        return sc_kernel(x, indices.astype(jnp.int32).T)

    return run


def kernel(x, indices):
    info = plsc.get_sparse_core_info()
    return _build(info.num_cores, info.num_subcores)(x, indices)
